# Initial kernel scaffold; baseline (speedup 1.0000x reference)
import jax
import jax.numpy as jnp
from jax.experimental import pallas as pl


def kernel(feat, edge_index, W_gnn, b_gnn, W_down, b_down, W_up, b_up):
    raise NotImplementedError("write your pallas kernel here")


# speedup vs baseline: 6.3939x; 6.3939x over previous
import functools

import jax
import jax.numpy as jnp
from jax import lax
from jax.experimental import pallas as pl
from jax.experimental.pallas import tpu as pltpu
from jax.experimental.pallas import tpu_sc as plsc

N = 10000
E = 320000
D = 128
BN = 64

NS = 16
BATCH = 128
NB_A = -(-E // (NS * BATCH))
NB_C = -(-(E // 2) // (NS * BATCH))
NPAD = 10240
DUMMY = 10016
ZROW = NPAD // NS
RB = 1000


def _deg_body(idx_hbm, ones_hbm, zeros_hbm, deg_out, idx_v, ones_v, z_v, deg_sp):
    c = lax.axis_index("c")
    s = lax.axis_index("s")
    pltpu.sync_copy(ones_hbm, ones_v)
    pltpu.sync_copy(zeros_hbm, z_v)
    pltpu.sync_copy(idx_hbm.at[c, s], idx_v)
    pltpu.sync_copy(z_v, deg_sp.at[pl.ds(s * ZROW, ZROW)])
    plsc.subcore_barrier()

    def body(b, carry):
        pltpu.sync_copy(ones_v, deg_sp.at[idx_v.at[b]], add=True)
        return carry

    lax.fori_loop(0, NB_A, body, 0)
    plsc.subcore_barrier()
    pltpu.sync_copy(deg_sp.at[pl.ds(s * ZROW, ZROW)], z_v)
    pltpu.sync_copy(z_v, deg_out.at[c, pl.ds(s * ZROW, ZROW)])


_deg_kernel = functools.partial(
    pl.kernel,
    out_type=jax.ShapeDtypeStruct((2, NPAD), jnp.float32),
    mesh=plsc.VectorSubcoreMesh(core_axis_name="c", subcore_axis_name="s"),
    scratch_types=[
        pltpu.VMEM((NB_A, BATCH), jnp.int32),
        pltpu.VMEM((BATCH,), jnp.float32),
        pltpu.VMEM((ZROW,), jnp.float32),
        pltpu.VMEM_SHARED((NPAD,), jnp.float32),
    ],
)(_deg_body)


def _agg_body(h_hbm, src_hbm, dst_hbm, zeros_hbm, out_hbm,
              src_v, dst_v, rows_v, agg_sp, sem):
    c = lax.axis_index("c")
    s = lax.axis_index("s")
    pltpu.sync_copy(src_hbm.at[c, s], src_v)
    pltpu.sync_copy(dst_hbm.at[c, s], dst_v)
    pltpu.sync_copy(zeros_hbm, rows_v)
    for k in range(ZROW // BATCH):
        pltpu.sync_copy(rows_v, agg_sp.at[pl.ds(s * ZROW + k * BATCH, BATCH)])
    plsc.subcore_barrier()

    def body(b, carry):
        pltpu.async_copy(h_hbm.at[src_v.at[b]], rows_v, sem).wait()
        pltpu.sync_copy(rows_v, agg_sp.at[dst_v.at[b]], add=True)
        return carry

    lax.fori_loop(0, NB_C, body, 0)
    plsc.subcore_barrier()
    for k in range(ZROW // BATCH):
        r0 = s * ZROW + k * BATCH
        pltpu.sync_copy(agg_sp.at[pl.ds(r0, BATCH)], rows_v)
        pltpu.sync_copy(rows_v, out_hbm.at[c, pl.ds(r0, BATCH), :])


_agg_kernel = functools.partial(
    pl.kernel,
    out_type=jax.ShapeDtypeStruct((2, NPAD, D), jnp.float32),
    mesh=plsc.VectorSubcoreMesh(core_axis_name="c", subcore_axis_name="s"),
    scratch_types=[
        pltpu.VMEM((NB_C, BATCH), jnp.int32),
        pltpu.VMEM((NB_C, BATCH), jnp.int32),
        pltpu.VMEM((BATCH, D), jnp.float32),
        pltpu.VMEM_SHARED((NPAD, D), jnp.float32),
        pltpu.SemaphoreType.DMA,
    ],
)(_agg_body)


def _norm_from_deg(deg_col):
    return jnp.where(deg_col > 0, lax.rsqrt(jnp.maximum(deg_col, 1e-12)), 0.0)


def _h_body(feat_ref, w_ref, deg_ref, h_ref):
    norm = _norm_from_deg(deg_ref[:, 0:1])
    h_ref[...] = jnp.dot(feat_ref[...] * norm, w_ref[...],
                         preferred_element_type=jnp.float32)


def _h_kernel(feat, w_gnn, deg_src):
    return pl.pallas_call(
        _h_body,
        grid=(N // RB,),
        in_specs=[
            pl.BlockSpec((RB, D), lambda i: (i, 0)),
            pl.BlockSpec((D, D), lambda i: (0, 0)),
            pl.BlockSpec((RB, 2), lambda i: (i, 0)),
        ],
        out_specs=pl.BlockSpec((RB, D), lambda i: (i, 0)),
        out_shape=jax.ShapeDtypeStruct((N, D), jnp.float32),
    )(feat, w_gnn, deg_src)


def _adapter_body(agg_ref, deg_ref, bg_ref, wd_ref, bd_ref, wu_ref, bu_ref,
                  y_ref):
    agg = agg_ref[0] + agg_ref[1]
    norm = _norm_from_deg(deg_ref[:, 1:2])
    out = agg * norm + bg_ref[0:1, :]
    a = jnp.dot(out, wd_ref[...], preferred_element_type=jnp.float32)
    a = jax.nn.gelu(a + bd_ref[0:1, :])
    a = jnp.dot(a, wu_ref[...], preferred_element_type=jnp.float32)
    y_ref[...] = a + bu_ref[0:1, :] + out


def _adapter_kernel(agg2, deg_dst, b_gnn, w_down, b_down, w_up, b_up):
    return pl.pallas_call(
        _adapter_body,
        grid=(N // RB,),
        in_specs=[
            pl.BlockSpec((2, RB, D), lambda i: (0, i, 0)),
            pl.BlockSpec((RB, 2), lambda i: (i, 0)),
            pl.BlockSpec((8, D), lambda i: (0, 0)),
            pl.BlockSpec((D, BN), lambda i: (0, 0)),
            pl.BlockSpec((8, BN), lambda i: (0, 0)),
            pl.BlockSpec((BN, D), lambda i: (0, 0)),
            pl.BlockSpec((8, D), lambda i: (0, 0)),
        ],
        out_specs=pl.BlockSpec((RB, D), lambda i: (i, 0)),
        out_shape=jax.ShapeDtypeStruct((N, D), jnp.float32),
    )(agg2, deg_dst, b_gnn, w_down, b_down, w_up, b_up)


def _pad_tiles(x, nb, fill):
    padn = NS * nb * BATCH - x.shape[0]
    return jnp.concatenate(
        [x, jnp.full((padn,), fill, jnp.int32)]).reshape(NS, nb, BATCH)


def kernel(feat, edge_index, W_gnn, b_gnn, W_down, b_down, W_up, b_up):
    src = edge_index[0].astype(jnp.int32)
    dst = edge_index[1].astype(jnp.int32)

    deg_idx = jnp.stack([_pad_tiles(src, NB_A, DUMMY),
                         _pad_tiles(dst, NB_A, DUMMY)])

    half = E // 2
    src_c = jnp.stack([_pad_tiles(src[:half], NB_C, 0),
                       _pad_tiles(src[half:], NB_C, 0)])
    dst_c = jnp.stack([_pad_tiles(dst[:half], NB_C, DUMMY),
                       _pad_tiles(dst[half:], NB_C, DUMMY)])

    ones1 = jnp.ones((BATCH,), jnp.float32)
    zeros1 = jnp.zeros((ZROW,), jnp.float32)
    zrows = jnp.zeros((BATCH, D), jnp.float32)

    degs = _deg_kernel(deg_idx, ones1, zeros1)
    degs_t = jnp.transpose(degs)

    h = _h_kernel(feat, W_gnn, degs_t)

    agg2 = _agg_kernel(h, src_c, dst_c, zrows)[:, :N, :]

    y = _adapter_kernel(
        agg2, degs_t,
        jnp.broadcast_to(b_gnn, (8, D)),
        W_down,
        jnp.broadcast_to(b_down, (8, BN)),
        W_up,
        jnp.broadcast_to(b_up, (8, D)),
    )
    return y
